# baseline (speedup 1.0000x reference)
import functools

import jax
import jax.numpy as jnp
from jax import lax
from jax.experimental import pallas as pl
from jax.experimental.pallas import tpu as pltpu
from jax.experimental.pallas import tpu_sc as plsc

_N = 320000
_F = 128
_S = 512
_EPS = 0.001

_B = 6400
_NB = _N // _B
_W = 32
_SPAD = 640

_NC = 2
_NS = 16
_NWORK = _NC * _NS
_RPW = _N // _NWORK
_C = 400
_NCH = _RPW // _C
_SUB = 80
_NSUB = _C // _SUB
_TROWS = _SPAD // _NS


def _sc_stats(x, seg):
    mesh = plsc.VectorSubcoreMesh(core_axis_name="c", subcore_axis_name="s")
    kernel_fn = pl.kernel(
        _sc_stats_kernel,
        out_type=[
            jax.ShapeDtypeStruct((_NC, _SPAD, _F), jnp.float32),
            jax.ShapeDtypeStruct((_NC, _SPAD, _F), jnp.float32),
            jax.ShapeDtypeStruct((_NC, _SPAD, 16), jnp.float32),
        ],
        mesh=mesh,
        scratch_types=[
            pltpu.VMEM((_C, _F), jnp.float32),
            pltpu.VMEM((_SUB,), jnp.int32),
            pltpu.VMEM((_SUB, 16), jnp.float32),
            pltpu.VMEM((_TROWS, _F), jnp.float32),
            pltpu.VMEM((_TROWS, 16), jnp.float32),
            pltpu.VMEM_SHARED((_SPAD, _F), jnp.float32),
            pltpu.VMEM_SHARED((_SPAD, _F), jnp.float32),
            pltpu.VMEM_SHARED((_SPAD, 16), jnp.float32),
        ],
    )
    return kernel_fn(x, seg)


def _sc_stats_kernel(x_hbm, seg_hbm, sum_hbm, sq_hbm, cnt_hbm,
                     xbuf, idxbuf, onesbuf, zbuf, zcnt,
                     sum_sh, sq_sh, cnt_sh):
    cid = lax.axis_index("c")
    sid = lax.axis_index("s")
    wid = cid * _NS + sid
    row0 = wid * _RPW

    @pl.loop(0, _TROWS)
    def _zrow(r):
        for f in range(_F // 16):
            zbuf[r, pl.ds(f * 16, 16)] = jnp.zeros((16,), jnp.float32)
        zcnt[r, :] = jnp.zeros((16,), jnp.float32)

    @pl.loop(0, _SUB)
    def _orow(r):
        onesbuf[r, :] = jnp.ones((16,), jnp.float32)

    t0 = sid * _TROWS
    pltpu.sync_copy(zbuf, sum_sh.at[pl.ds(t0, _TROWS)])
    pltpu.sync_copy(zbuf, sq_sh.at[pl.ds(t0, _TROWS)])
    pltpu.sync_copy(zcnt, cnt_sh.at[pl.ds(t0, _TROWS)])
    plsc.subcore_barrier()

    @pl.loop(0, _NCH)
    def _chunk(k):
        base = row0 + k * _C
        pltpu.sync_copy(x_hbm.at[pl.ds(base, _C)], xbuf)

        for j in range(_NSUB):
            pltpu.sync_copy(seg_hbm.at[pl.ds(base + j * _SUB, _SUB)], idxbuf)
            pltpu.sync_copy(xbuf.at[pl.ds(j * _SUB, _SUB)],
                            sum_sh.at[idxbuf], add=True)
            pltpu.sync_copy(onesbuf, cnt_sh.at[idxbuf], add=True)

        @pl.loop(0, _C)
        def _sqrow(r):
            for f in range(_F // 16):
                v = xbuf[r, pl.ds(f * 16, 16)]
                xbuf[r, pl.ds(f * 16, 16)] = v * v

        for j in range(_NSUB):
            pltpu.sync_copy(seg_hbm.at[pl.ds(base + j * _SUB, _SUB)], idxbuf)
            pltpu.sync_copy(xbuf.at[pl.ds(j * _SUB, _SUB)],
                            sq_sh.at[idxbuf], add=True)

    plsc.subcore_barrier()

    pltpu.sync_copy(sum_sh.at[pl.ds(t0, _TROWS)],
                    sum_hbm.at[cid, pl.ds(t0, _TROWS)])
    pltpu.sync_copy(sq_sh.at[pl.ds(t0, _TROWS)],
                    sq_hbm.at[cid, pl.ds(t0, _TROWS)])
    pltpu.sync_copy(cnt_sh.at[pl.ds(t0, _TROWS)],
                    cnt_hbm.at[cid, pl.ds(t0, _TROWS)])


def _finalize_kernel(sum_ref, sq_ref, cnt_ref, table_ref):
    sums = sum_ref[0] + sum_ref[1]
    sqs = sq_ref[0] + sq_ref[1]
    cnt = (cnt_ref[0] + cnt_ref[1])[:, 0:1]
    mean = sums / jnp.maximum(cnt, 1.0)
    ssq = (jnp.sum(sqs, axis=1, keepdims=True)
           - cnt * jnp.sum(mean * mean, axis=1, keepdims=True))
    var = ssq / (cnt * jnp.float32(_F) - 1.0)
    invstd = lax.rsqrt(var + _EPS)
    table_ref[:, 0:_F] = mean.astype(jnp.bfloat16)
    table_ref[:, _F:2 * _F] = jnp.broadcast_to(
        invstd, (_SPAD, _F)).astype(jnp.bfloat16)


def _norm_kernel(x_ref, seg_ref, table_ref, out_ref):
    seg = seg_ref[0, 0, :]
    s0 = (seg[0] // 16) * 16
    smax = seg[_B - 1]
    x = x_ref[...]

    narrow = (smax - s0) < _W

    @pl.when(narrow)
    def _narrow():
        win = table_ref[pl.ds(s0, _W), :]
        col = lax.broadcasted_iota(jnp.int32, (_B, _W), 1)
        oh = (seg[:, None] - s0 == col).astype(
            jnp.float32).astype(jnp.bfloat16)
        rows = lax.dot_general(
            oh, win, (((1,), (0,)), ((), ())),
            preferred_element_type=jnp.float32)
        out_ref[...] = (x - rows[:, 0:_F]) * rows[:, _F:2 * _F]

    @pl.when(jnp.logical_not(narrow))
    def _wide():
        win = table_ref[pl.ds(0, _S), :]
        col = lax.broadcasted_iota(jnp.int32, (_B, _S), 1)
        oh = (seg[:, None] == col).astype(
            jnp.float32).astype(jnp.bfloat16)
        rows = lax.dot_general(
            oh, win, (((1,), (0,)), ((), ())),
            preferred_element_type=jnp.float32)
        out_ref[...] = (x - rows[:, 0:_F]) * rows[:, _F:2 * _F]


def kernel(x, i):
    seg = i.astype(jnp.int32)
    seg3 = seg.reshape(_NB, 1, _B)

    sums, sqs, cnts = _sc_stats(x, seg)

    table = pl.pallas_call(
        _finalize_kernel,
        in_specs=[
            pl.BlockSpec((_NC, _SPAD, _F), lambda: (0, 0, 0)),
            pl.BlockSpec((_NC, _SPAD, _F), lambda: (0, 0, 0)),
            pl.BlockSpec((_NC, _SPAD, 16), lambda: (0, 0, 0)),
        ],
        out_specs=pl.BlockSpec((_SPAD, 2 * _F), lambda: (0, 0)),
        out_shape=jax.ShapeDtypeStruct((_SPAD, 2 * _F), jnp.bfloat16),
    )(sums, sqs, cnts)

    out = pl.pallas_call(
        _norm_kernel,
        grid=(_NB,),
        in_specs=[
            pl.BlockSpec((_B, _F), lambda b: (b, 0)),
            pl.BlockSpec((1, 1, _B), lambda b: (b, 0, 0)),
            pl.BlockSpec((_SPAD, 2 * _F), lambda b: (0, 0)),
        ],
        out_specs=pl.BlockSpec((_B, _F), lambda b: (b, 0)),
        out_shape=jax.ShapeDtypeStruct((_N, _F), jnp.float32),
    )(x, seg3, table)

    return out
